# Initial kernel scaffold; baseline (speedup 1.0000x reference)
import jax
import jax.numpy as jnp
from jax.experimental import pallas as pl


def kernel(x, lengths):
    raise NotImplementedError("write your pallas kernel here")


# speedup vs baseline: 2.7989x; 2.7989x over previous
import functools

import jax
import jax.numpy as jnp
from jax import lax
from jax.experimental import pallas as pl
from jax.experimental.pallas import tpu as pltpu
from jax.experimental.pallas import tpu_sc as plsc

_BLANK = 0
_S, _B, _V = 2048, 16, 1024
_L = 16


_R = 1024
_NB = (_S * _B) // _R


def _argmax_body(x_ref, o_ref):
    xb = x_ref[...]
    m = jnp.max(xb, axis=1, keepdims=True)
    lane = lax.broadcasted_iota(jnp.int32, xb.shape, 1)
    idx = jnp.min(jnp.where(xb == m, lane, _V), axis=1)
    o_ref[...] = idx.reshape(1, 1, _R).astype(jnp.int32)


_argmax_call = pl.pallas_call(
    _argmax_body,
    grid=(_NB,),
    in_specs=[pl.BlockSpec((_R, _V), lambda i: (i, 0))],
    out_specs=pl.BlockSpec((1, 1, _R), lambda i: (i, 0, 0)),
    out_shape=jax.ShapeDtypeStruct((_NB, 1, _R), jnp.int32),
)


@functools.partial(
    pl.kernel,
    out_type=[
        jax.ShapeDtypeStruct((_B, _S), jnp.int32),
        jax.ShapeDtypeStruct((_B, _L), jnp.int32),
    ],
    mesh=plsc.VectorSubcoreMesh(core_axis_name="c", subcore_axis_name="s"),
    compiler_params=pltpu.CompilerParams(needs_layout_passes=False),
    scratch_types=[
        pltpu.VMEM((_S,), jnp.int32),
        pltpu.VMEM((_L,), jnp.int32),
        pltpu.VMEM((_S,), jnp.int32),
        pltpu.VMEM((_L,), jnp.int32),
    ],
)
def _sc_decode(ml_hbm, len_hbm, tok_hbm, cnt_hbm, buf_v, len_v, out_v, cnt_v):
    c = lax.axis_index("c")
    s = lax.axis_index("s")
    wid = s * 2 + c

    @pl.when(wid < _B)
    def _():
        pltpu.sync_copy(ml_hbm.at[wid], buf_v)
        pltpu.sync_copy(len_hbm, len_v)
        iota = lax.iota(jnp.int32, _L)
        length = jnp.sum(jnp.where(iota == wid, len_v[...], 0))

        neg1 = jnp.full((_L,), -1, jnp.int32)

        def init_body(i, carry):
            out_v[pl.ds(i * _L, _L)] = neg1
            return carry

        lax.fori_loop(0, _S // _L, init_body, 0)

        rot_key = (iota + 1) % _L

        def body(i, carry):
            off, last = carry
            v = buf_v[pl.ds(i * _L, _L)]
            _sorted_k, rot = plsc.sort_key_val(rot_key, v)
            prev = jnp.where(iota == 0, last, rot)
            pos = i * _L + iota
            valid = pos < length
            keep = (v != _BLANK) & (v != prev) & valid
            k32 = keep.astype(jnp.int32)
            dest = off + plsc.cumsum(k32) - 1
            plsc.store_scatter(out_v, [dest], v, mask=keep)
            new_off = off + jnp.sum(k32)
            new_last = jnp.sum(jnp.where(iota == _L - 1, v, 0))
            return (new_off, new_last)

        nchunks = (length + _L - 1) // _L
        total, _unused = lax.fori_loop(
            0, nchunks, body, (jnp.int32(0), jnp.int32(-1)))
        pltpu.sync_copy(out_v, tok_hbm.at[wid])
        cnt_v[...] = jnp.zeros((_L,), jnp.int32) + total
        pltpu.sync_copy(cnt_v, cnt_hbm.at[wid])


def kernel(x, lengths):
    ml = _argmax_call(x.reshape(_S * _B, _V))
    ml_bs = ml.reshape(_S, _B).T
    tokens, counts2d = _sc_decode(ml_bs, lengths.astype(jnp.int32))
    return tokens, counts2d[:, 0]


# speedup vs baseline: 3.4549x; 1.2344x over previous
import functools

import jax
import jax.numpy as jnp
from jax import lax
from jax.experimental import pallas as pl
from jax.experimental.pallas import tpu as pltpu
from jax.experimental.pallas import tpu_sc as plsc

_BLANK = 0
_S, _B, _V = 2048, 16, 1024
_L = 16


_BS = 128
_NB = _S // _BS


def _argmax_body(x_ref, o_ref):
    xb = x_ref[...]
    m = jnp.max(xb, axis=2, keepdims=True)
    lane = lax.broadcasted_iota(jnp.int32, xb.shape, 2)
    idx = jnp.min(jnp.where(xb == m, lane, _V), axis=2)
    o_ref[...] = idx.astype(jnp.int32).T


_argmax_call = pl.pallas_call(
    _argmax_body,
    grid=(_NB,),
    in_specs=[pl.BlockSpec((_BS, _B, _V), lambda i: (i, 0, 0))],
    out_specs=pl.BlockSpec((_B, _BS), lambda i: (0, i)),
    out_shape=jax.ShapeDtypeStruct((_B, _S), jnp.int32),
)


@functools.partial(
    pl.kernel,
    out_type=[
        jax.ShapeDtypeStruct((_B, _S), jnp.int32),
        jax.ShapeDtypeStruct((_B, _L), jnp.int32),
    ],
    mesh=plsc.VectorSubcoreMesh(core_axis_name="c", subcore_axis_name="s"),
    compiler_params=pltpu.CompilerParams(needs_layout_passes=False),
    scratch_types=[
        pltpu.VMEM((_S,), jnp.int32),
        pltpu.VMEM((_L,), jnp.int32),
        pltpu.VMEM((_S,), jnp.int32),
        pltpu.VMEM((_L,), jnp.int32),
    ],
)
def _sc_decode(ml_hbm, len_hbm, tok_hbm, cnt_hbm, buf_v, len_v, out_v, cnt_v):
    c = lax.axis_index("c")
    s = lax.axis_index("s")
    wid = s * 2 + c

    @pl.when(wid < _B)
    def _():
        pltpu.sync_copy(ml_hbm.at[wid], buf_v)
        pltpu.sync_copy(len_hbm, len_v)
        iota = lax.iota(jnp.int32, _L)
        length = jnp.sum(jnp.where(iota == wid, len_v[...], 0))

        neg1 = jnp.full((_L,), -1, jnp.int32)

        def init_body(i, carry):
            out_v[pl.ds(i * _L, _L)] = neg1
            return carry

        lax.fori_loop(0, _S // _L, init_body, 0)

        rot_key = (iota + 1) % _L

        def body(i, carry):
            off, last = carry
            v = buf_v[pl.ds(i * _L, _L)]
            _sorted_k, rot = plsc.sort_key_val(rot_key, v)
            prev = jnp.where(iota == 0, last, rot)
            pos = i * _L + iota
            valid = pos < length
            keep = (v != _BLANK) & (v != prev) & valid
            k32 = keep.astype(jnp.int32)
            dest = off + plsc.cumsum(k32) - 1
            plsc.store_scatter(out_v, [dest], v, mask=keep)
            new_off = off + jnp.sum(k32)
            new_last = jnp.sum(jnp.where(iota == _L - 1, v, 0))
            return (new_off, new_last)

        nchunks = (length + _L - 1) // _L
        total, _unused = lax.fori_loop(
            0, nchunks, body, (jnp.int32(0), jnp.int32(-1)))
        pltpu.sync_copy(out_v, tok_hbm.at[wid])
        cnt_v[...] = jnp.zeros((_L,), jnp.int32) + total
        pltpu.sync_copy(cnt_v, cnt_hbm.at[wid])


def kernel(x, lengths):
    ml_bs = _argmax_call(x)
    tokens, counts2d = _sc_decode(ml_bs, lengths.astype(jnp.int32))
    return tokens, counts2d[:, 0]


# speedup vs baseline: 3.6714x; 1.0627x over previous
import functools

import jax
import jax.numpy as jnp
from jax import lax
from jax.experimental import pallas as pl
from jax.experimental.pallas import tpu as pltpu
from jax.experimental.pallas import tpu_sc as plsc

_BLANK = 0
_S, _B, _V = 2048, 16, 1024
_L = 16


_BS = 256
_NB = _S // _BS


def _argmax_body(x_ref, o_ref):
    xb = x_ref[...]
    m = jnp.max(xb, axis=2, keepdims=True)
    lane = lax.broadcasted_iota(jnp.int32, xb.shape, 2)
    idx = jnp.min(jnp.where(xb == m, lane, _V), axis=2)
    o_ref[...] = idx.astype(jnp.int32).T


_argmax_call = pl.pallas_call(
    _argmax_body,
    grid=(_NB,),
    in_specs=[pl.BlockSpec((_BS, _B, _V), lambda i: (i, 0, 0))],
    out_specs=pl.BlockSpec((_B, _BS), lambda i: (0, i)),
    out_shape=jax.ShapeDtypeStruct((_B, _S), jnp.int32),
)


@functools.partial(
    pl.kernel,
    out_type=[
        jax.ShapeDtypeStruct((_B, _S), jnp.int32),
        jax.ShapeDtypeStruct((_B,), jnp.int32),
    ],
    mesh=plsc.VectorSubcoreMesh(core_axis_name="c", subcore_axis_name="s"),
    compiler_params=pltpu.CompilerParams(needs_layout_passes=False),
    scratch_types=[
        pltpu.VMEM((_S,), jnp.int32),
        pltpu.VMEM((_L,), jnp.int32),
        pltpu.VMEM((_S,), jnp.int32),
        pltpu.VMEM((_L,), jnp.int32),
        pltpu.VMEM((_L, _L), jnp.int32),
        pltpu.VMEM_SHARED((_L, _L), jnp.int32),
    ],
)
def _sc_decode(ml_hbm, len_hbm, tok_hbm, cnt_hbm,
               buf_v, len_v, out_v, cnt_v, cnt_mat_v, shared_cnt):
    c = lax.axis_index("c")
    s = lax.axis_index("s")

    @pl.when(c == 0)
    def _():
        iota = lax.iota(jnp.int32, _L)
        zero_v = jnp.zeros((_L,), jnp.int32)
        neg1 = jnp.full((_L,), -1, jnp.int32)

        pltpu.sync_copy(ml_hbm.at[s], buf_v)
        pltpu.sync_copy(len_hbm, len_v)
        length = jnp.sum(jnp.where(iota == s, len_v[...], 0))

        def init_body(i, carry):
            out_v[pl.ds(i * _L, _L)] = neg1
            return carry

        lax.fori_loop(0, _S // _L, init_body, 0)

        def body(i, off_vec):
            base = i * _L
            pos = base + iota
            v = buf_v[pl.ds(base, _L)]
            pgather = plsc.load_gather(buf_v, [jnp.maximum(pos - 1, 0)])
            prev = jnp.where(pos == 0, -1, pgather)
            valid = pos < length
            keep = (v != _BLANK) & (v != prev) & valid
            dest = off_vec + plsc.cumsum(keep.astype(jnp.int32)) - 1
            plsc.store_scatter(out_v, [dest], v, mask=keep)
            return off_vec + plsc.all_reduce_population_count(keep)

        nchunks = (length + _L - 1) // _L
        total_vec = lax.fori_loop(0, nchunks, body, zero_v)
        pltpu.sync_copy(out_v, tok_hbm.at[s])

        cnt_v[...] = total_vec
        pltpu.sync_copy(cnt_v, shared_cnt.at[s])
        plsc.subcore_barrier()

        @pl.when(s == 0)
        def _():
            pltpu.sync_copy(shared_cnt, cnt_mat_v)
            diag = plsc.load_gather(cnt_mat_v, [iota, iota])
            cnt_v[...] = diag
            pltpu.sync_copy(cnt_v, cnt_hbm)


def kernel(x, lengths):
    ml_bs = _argmax_call(x)
    tokens, counts = _sc_decode(ml_bs, lengths.astype(jnp.int32))
    return tokens, counts
